# baseline (device time: 180028 ns/iter reference)
import jax
import jax.numpy as jnp
from jax import lax
from jax.experimental import pallas as pl
from jax.experimental.pallas import tpu as pltpu

T_LOC = 1024
T_GLB = 2048
D = 1024
F = 4096
E = 16
E_LOC = 8
K = 2
NPAIR = T_GLB * K
CAP = 320
NSLOT = E_LOC * CAP
FB = 2048


def _peer():
    ix = lax.axis_index("x")
    iy = lax.axis_index("y")
    iz = lax.axis_index("z")
    return ix, (1 - ix, iy, iz)


def _xpeer_barrier(peer):
    barrier = pltpu.get_barrier_semaphore()
    pl.semaphore_signal(
        barrier, inc=1, device_id=peer, device_id_type=pl.DeviceIdType.MESH
    )
    pl.semaphore_wait(barrier, 1)


def _ag_body(x_ref, rme_ref, xall_ref, gall_ref, rpeer_ref, send_sems, recv_sems):
    ix, peer = _peer()
    _xpeer_barrier(peer)

    my_off = ix * T_LOC

    r_rdma = pltpu.make_async_remote_copy(
        src_ref=rme_ref,
        dst_ref=rpeer_ref,
        send_sem=send_sems.at[0],
        recv_sem=recv_sems.at[0],
        device_id=peer,
        device_id_type=pl.DeviceIdType.MESH,
    )
    r_rdma.start()

    xall_ref[pl.ds(my_off, T_LOC), :] = x_ref[...].astype(jnp.bfloat16)
    x_rdma = pltpu.make_async_remote_copy(
        src_ref=xall_ref.at[pl.ds(my_off, T_LOC), :],
        dst_ref=xall_ref.at[pl.ds(my_off, T_LOC), :],
        send_sem=send_sems.at[1],
        recv_sem=recv_sems.at[1],
        device_id=peer,
        device_id_type=pl.DeviceIdType.MESH,
    )
    x_rdma.start()

    gme = jnp.dot(
        x_ref[...],
        rme_ref[...],
        preferred_element_type=jnp.float32,
        precision=lax.Precision.HIGHEST,
    )
    r_rdma.wait_recv()
    gpe = jnp.dot(
        x_ref[...],
        rpeer_ref[...],
        preferred_element_type=jnp.float32,
        precision=lax.Precision.HIGHEST,
    )
    cols = jnp.where(
        ix == 0,
        jnp.concatenate([gme, gpe], axis=1),
        jnp.concatenate([gpe, gme], axis=1),
    )
    gall_ref[pl.ds(my_off, T_LOC), :] = cols
    g_rdma = pltpu.make_async_remote_copy(
        src_ref=gall_ref.at[pl.ds(my_off, T_LOC), :],
        dst_ref=gall_ref.at[pl.ds(my_off, T_LOC), :],
        send_sem=send_sems.at[2],
        recv_sem=recv_sems.at[2],
        device_id=peer,
        device_id_type=pl.DeviceIdType.MESH,
    )
    g_rdma.start()

    r_rdma.wait_send()
    x_rdma.wait()
    g_rdma.wait()


def _ag_call(x, router):
    return pl.pallas_call(
        _ag_body,
        out_shape=[
            jax.ShapeDtypeStruct((T_GLB, D), jnp.bfloat16),
            jax.ShapeDtypeStruct((T_GLB, E), jnp.float32),
        ],
        in_specs=[
            pl.BlockSpec(memory_space=pltpu.VMEM),
            pl.BlockSpec(memory_space=pltpu.VMEM),
        ],
        out_specs=[
            pl.BlockSpec(memory_space=pltpu.VMEM),
            pl.BlockSpec(memory_space=pltpu.VMEM),
        ],
        scratch_shapes=[
            pltpu.VMEM((T_LOC, E_LOC), jnp.float32),
            pltpu.SemaphoreType.DMA((3,)),
            pltpu.SemaphoreType.DMA((3,)),
        ],
        compiler_params=pltpu.CompilerParams(collective_id=0),
    )(x, router)


def _ffn_body(
    xall_ref, s0_ref, s1_ref, w0_ref, w1r_ref, w1_ref, w2_ref, out_ref, xg_ref, acc_ref
):
    e = pl.program_id(0)
    f = pl.program_id(1)
    nf = pl.num_programs(1)

    @pl.when(f == 0)
    def _():
        slot_id = lax.broadcasted_iota(jnp.int32, (CAP, T_GLB), 0) + e * CAP
        m0 = slot_id == s0_ref[...]
        m1 = slot_id == s1_ref[...]
        ohw = jnp.where(m0, w0_ref[...], 0.0) + jnp.where(m1, w1r_ref[...], 0.0)
        xg_ref[...] = jnp.dot(
            ohw.astype(jnp.bfloat16),
            xall_ref[...],
            preferred_element_type=jnp.float32,
        ).astype(jnp.bfloat16)

    h = jnp.maximum(
        jnp.dot(
            xg_ref[...],
            w1_ref[0].astype(jnp.bfloat16),
            preferred_element_type=jnp.float32,
        ),
        0.0,
    )
    y = jnp.dot(
        h.astype(jnp.bfloat16),
        w2_ref[0].astype(jnp.bfloat16),
        preferred_element_type=jnp.float32,
    )

    @pl.when(f == 0)
    def _():
        acc_ref[...] = y

    @pl.when(f != 0)
    def _():
        acc_ref[...] += y

    @pl.when(f == nf - 1)
    def _():
        out_ref[0] = acc_ref[...].astype(jnp.bfloat16)


def _ffn_call(x_all, s0, s1, w0, w1r, W1, W2):
    return pl.pallas_call(
        _ffn_body,
        grid=(E_LOC, F // FB),
        out_shape=jax.ShapeDtypeStruct((E_LOC, CAP, D), jnp.bfloat16),
        in_specs=[
            pl.BlockSpec((T_GLB, D), lambda e, f: (0, 0)),
            pl.BlockSpec((1, T_GLB), lambda e, f: (0, 0)),
            pl.BlockSpec((1, T_GLB), lambda e, f: (0, 0)),
            pl.BlockSpec((1, T_GLB), lambda e, f: (0, 0)),
            pl.BlockSpec((1, T_GLB), lambda e, f: (0, 0)),
            pl.BlockSpec((1, D, FB), lambda e, f: (e, 0, f)),
            pl.BlockSpec((1, FB, D), lambda e, f: (e, f, 0)),
        ],
        out_specs=pl.BlockSpec((1, CAP, D), lambda e, f: (e, 0, 0)),
        scratch_shapes=[
            pltpu.VMEM((CAP, D), jnp.bfloat16),
            pltpu.VMEM((CAP, D), jnp.float32),
        ],
        compiler_params=pltpu.CompilerParams(
            dimension_semantics=("parallel", "arbitrary"),
            vmem_limit_bytes=60 * 1024 * 1024,
        ),
    )(x_all, s0, s1, w0, w1r, W1, W2)


RS_NCH = 4
RS_CH = T_LOC // RS_NCH


def _rs_body(yg_ref, sp_ref, out_ref, sbuf_ref, rbuf_ref, send_sems, recv_sems):
    ix, peer = _peer()
    _xpeer_barrier(peer)

    def part_rows(off, n):
        s0 = sp_ref[pl.ds(off, n), 0:1]
        s1 = sp_ref[pl.ds(off, n), 1:2]
        slot_id = lax.broadcasted_iota(jnp.int32, (n, NSLOT), 1)
        oh = ((slot_id == s0) | (slot_id == s1)).astype(jnp.bfloat16)
        return jnp.dot(oh, yg_ref[...], preferred_element_type=jnp.float32)

    rdmas = []
    for c in range(RS_NCH):
        sbuf_ref[pl.ds(c * RS_CH, RS_CH), :] = part_rows(
            (1 - ix) * T_LOC + c * RS_CH, RS_CH
        ).astype(jnp.bfloat16)
        rdma = pltpu.make_async_remote_copy(
            src_ref=sbuf_ref.at[pl.ds(c * RS_CH, RS_CH), :],
            dst_ref=rbuf_ref.at[pl.ds(c * RS_CH, RS_CH), :],
            send_sem=send_sems.at[c],
            recv_sem=recv_sems.at[c],
            device_id=peer,
            device_id_type=pl.DeviceIdType.MESH,
        )
        rdma.start()
        rdmas.append(rdma)
    out_ref[...] = part_rows(ix * T_LOC, T_LOC)
    for rdma in rdmas:
        rdma.wait()
    out_ref[...] += rbuf_ref[...].astype(jnp.float32)


def _rs_call(yg, sp):
    return pl.pallas_call(
        _rs_body,
        out_shape=jax.ShapeDtypeStruct((T_LOC, D), jnp.float32),
        in_specs=[
            pl.BlockSpec(memory_space=pltpu.VMEM),
            pl.BlockSpec(memory_space=pltpu.VMEM),
        ],
        out_specs=pl.BlockSpec(memory_space=pltpu.VMEM),
        scratch_shapes=[
            pltpu.VMEM((T_LOC, D), jnp.bfloat16),
            pltpu.VMEM((T_LOC, D), jnp.bfloat16),
            pltpu.SemaphoreType.DMA((RS_NCH,)),
            pltpu.SemaphoreType.DMA((RS_NCH,)),
        ],
        compiler_params=pltpu.CompilerParams(collective_id=1),
    )(yg, sp)


def kernel(x, router, W1, W2):
    ix = lax.axis_index("x")

    x_all, gates = _ag_call(x, router)

    tv, ti = lax.top_k(gates, K)
    w = jax.nn.softmax(tv, axis=1)
    flat_e = ti.reshape(-1)

    le = flat_e - E_LOC * ix
    local = (le >= 0) & (le < E_LOC)

    oh = (le[:, None] == jnp.arange(E_LOC)[None, :]) & local[:, None]
    pos = jnp.cumsum(oh.astype(jnp.int32), axis=0)
    rank = jnp.sum(jnp.where(oh, pos - 1, 0), axis=1)
    slot = jnp.where(local & (rank < CAP), le * CAP + rank, NSLOT).astype(jnp.int32)

    sp = slot.reshape(T_GLB, K)
    s0 = sp[:, 0].reshape(1, T_GLB)
    s1 = sp[:, 1].reshape(1, T_GLB)
    w0 = w[:, 0].reshape(1, T_GLB)
    w1r = w[:, 1].reshape(1, T_GLB)

    yg = _ffn_call(x_all, s0, s1, w0, w1r, W1, W2)

    return _rs_call(yg.reshape(NSLOT, D), sp)
